# baseline (speedup 1.0000x reference)
import functools

import jax
import jax.numpy as jnp
import numpy as np
from jax import lax
from jax.experimental import pallas as pl
from jax.experimental.pallas import tpu as pltpu
from jax.experimental.pallas import tpu_sc as plsc

NC = 2
NSUB = 16
NW = NC * NSUB

CUTOFF = 5.0
NRAD = 6
NSPH = 7
ENV_P = 5
HID = 128
INT_EMB = 64
BASIS = 8
OUT_EMB = 256
N_G = 64

BE = 2000
BT = 4096
BN = 2000
BNV = 2048

TPAD = 655360
NPAD = 10240
WIN = 8000
NCHUNK = 40
WACC = 8064

_INTERPRET = False


def _swish(x):
    return x / (1.0 + jnp.exp(-x))


def _mm(a, b):
    return jnp.dot(a.astype(jnp.bfloat16), b.astype(jnp.bfloat16),
                   preferred_element_type=jnp.float32)


def _envelope(d):
    p = ENV_P
    a = -(p + 1) * (p + 2) / 2.0
    b = float(p * (p + 2))
    c = -p * (p + 1) / 2.0
    d2 = d * d
    d4 = d2 * d2
    d5 = d4 * d
    return 1.0 / d + a * d5 + b * d5 * d + c * d5 * d2


def _sin_ladder(d):
    s1 = jnp.sin(jnp.pi * d)
    c1 = jnp.cos(jnp.pi * d)
    sins = [s1]
    ck = c1
    for _ in range(NRAD - 1):
        sk = sins[-1]
        sins.append(sk * c1 + ck * s1)
        ck = ck * c1 - sk * s1
    return jnp.concatenate(sins, axis=1)


def _full(spec_shape):
    return pl.BlockSpec(spec_shape, lambda *_: tuple(0 for _ in spec_shape))


def _edge_init_body(ea_ref, dist_ref, we_ref, e1_ref, rbf_ref):
    e1_ref[...] = ea_ref[...] @ we_ref[...]
    d = dist_ref[...] / CUTOFF
    rbf_ref[...] = _envelope(d) * _sin_ladder(d)


def _edge_init(edge_attr, dist, we):
    E = edge_attr.shape[0]
    grid = (E // BE,)
    return pl.pallas_call(
        _edge_init_body,
        grid=grid,
        in_specs=[
            pl.BlockSpec((BE, 12), lambda i: (i, 0)),
            pl.BlockSpec((BE, 1), lambda i: (i, 0)),
            _full((12, HID)),
        ],
        out_specs=[
            pl.BlockSpec((BE, HID), lambda i: (i, 0)),
            pl.BlockSpec((BE, NRAD), lambda i: (i, 0)),
        ],
        out_shape=[
            jax.ShapeDtypeStruct((E, HID), jnp.float32),
            jax.ShapeDtypeStruct((E, NRAD), jnp.float32),
        ],
        interpret=_INTERPRET,
    )(edge_attr, dist[:, None], we)


def _node_init_body(x_ref, batch_ref, wn_ref, u_ref):
    v = x_ref[...] @ wn_ref[...]
    gid = lax.broadcasted_iota(jnp.int32, (N_G, v.shape[0]), 0)
    onehot = (gid == batch_ref[0]).astype(jnp.float32)

    @pl.when(pl.program_id(0) == 0)
    def _():
        u_ref[...] = jnp.zeros_like(u_ref)

    u_ref[...] += onehot @ v


def _node_init(x, batch, wn):
    N = x.shape[0]
    return pl.pallas_call(
        _node_init_body,
        grid=(N // BN,),
        in_specs=[
            pl.BlockSpec((BN, 48), lambda i: (i, 0)),
            pl.BlockSpec((1, 1, BN), lambda i: (i, 0, 0)),
            _full((48, HID)),
        ],
        out_specs=_full((N_G, HID)),
        out_shape=jax.ShapeDtypeStruct((N_G, HID), jnp.float32),
        interpret=_INTERPRET,
    )(x, batch.reshape(N // BN, 1, BN), wn)


_MESH = plsc.VectorSubcoreMesh(core_axis_name="c", subcore_axis_name="s",
                               num_cores=NC, num_subcores=NSUB)


def _sc_gather(xdn, idx_kj2d):
    per_tile = TPAD // NW
    ngrp = per_tile // (4 * 128)

    @functools.partial(
        pl.kernel,
        out_type=jax.ShapeDtypeStruct((TPAD, HID), jnp.float32),
        mesh=_MESH,
        scratch_types=[
            pltpu.VMEM((4, 128), jnp.int32),
            pltpu.VMEM((4, 128, HID), jnp.float32),
            pltpu.SemaphoreType.DMA, pltpu.SemaphoreType.DMA,
        ],
    )
    def k(xdn_hbm, kj_hbm, xg_hbm, idx_v, rows_v, gsem, ssem):
        wid = lax.axis_index("c") * NSUB + lax.axis_index("s")
        grp0 = wid * (ngrp * 4)

        def body(j, _):
            blk = grp0 + j * 4
            pltpu.sync_copy(kj_hbm.at[pl.ds(blk, 4)], idx_v)

            @pl.when(j > 0)
            def _():
                for b in range(4):
                    pltpu.make_async_copy(
                        xg_hbm.at[pl.ds(0, 128)], rows_v.at[b], ssem).wait()

            descs = [pltpu.async_copy(xdn_hbm.at[idx_v.at[b]], rows_v.at[b],
                                      gsem) for b in range(4)]
            for d in descs:
                d.wait()
            for b in range(4):
                pltpu.async_copy(rows_v.at[b],
                                 xg_hbm.at[pl.ds((blk + b) * 128, 128)], ssem)
            return 0

        lax.fori_loop(0, ngrp, body, 0)
        for b in range(4):
            pltpu.make_async_copy(xg_hbm.at[pl.ds(0, 128)], rows_v.at[b],
                                  ssem).wait()

    return k(xdn, idx_kj2d)


def _sc_tri_scatter(y, idx_ji_pad, bounds, zrows):
    zstripe = WACC // NSUB

    @functools.partial(
        pl.kernel,
        out_type=jax.ShapeDtypeStruct((NC, 320000 + 64, HID), jnp.float32),
        mesh=_MESH,
        scratch_types=[
            pltpu.VMEM_SHARED((WACC, HID), jnp.float32),
            pltpu.VMEM((zstripe // 3, HID), jnp.float32),
            pltpu.VMEM((128,), jnp.int32),
            pltpu.VMEM((128,), jnp.int32),
            pltpu.VMEM((128, HID), jnp.float32),
            pltpu.VMEM((NCHUNK, 16), jnp.int32),
        ],
    )
    def k(y_hbm, ji_hbm, bounds_hbm, z_hbm, out_hbm,
          acc, zbuf, idxj_v, idxc_v, rows_v, bounds_v):
        core = lax.axis_index("c")
        sid = lax.axis_index("s")
        wid = core * NSUB + sid
        pltpu.sync_copy(z_hbm, zbuf)
        pltpu.sync_copy(bounds_hbm, bounds_v)

        for c in range(NCHUNK):
            for z in range(3):
                pltpu.sync_copy(
                    zbuf,
                    acc.at[pl.ds(sid * zstripe + z * (zstripe // 3),
                                 zstripe // 3)])
            plsc.subcore_barrier()
            row = bounds_v[c]
            s = row[0]
            e = row[1]
            nb = lax.shift_right_logical(e - s + (NW * 128 - 1), 12)
            base = c * WIN

            def inner(j, _, s=s, base=base):
                off = pl.multiple_of(s + (j * NW + wid) * 128, 8)
                pltpu.sync_copy(ji_hbm.at[pl.ds(off, 128)], idxj_v)
                pltpu.sync_copy(y_hbm.at[pl.ds(off, 128)], rows_v)
                for m in range(8):
                    lv = idxj_v[pl.ds(m * 16, 16)] - base
                    inb = (lv >= 0) & (lv < WIN)
                    idxc_v[pl.ds(m * 16, 16)] = jnp.where(inb, lv, WIN)
                pltpu.sync_copy(rows_v, acc.at[idxc_v], add=True)
                return 0

            lax.fori_loop(0, nb, inner, 0)
            plsc.subcore_barrier()
            pltpu.sync_copy(
                acc.at[pl.ds(sid * zstripe, zstripe)],
                out_hbm.at[core, pl.ds(base + sid * zstripe, zstripe)])
            plsc.subcore_barrier()

    return k(y, idx_ji_pad, bounds, zrows)


def _sc_node_scatter(e2, i_idx, zrows):
    E = e2.shape[0]
    per_core = E // NC
    per_tile = per_core // NSUB
    BLK = 80
    nblk = per_tile // BLK
    stripe = NPAD // NSUB

    @functools.partial(
        pl.kernel,
        out_type=jax.ShapeDtypeStruct((NC, NPAD, HID), jnp.float32),
        mesh=_MESH,
        scratch_types=[
            pltpu.VMEM_SHARED((NPAD, HID), jnp.float32),
            pltpu.VMEM((BLK,), jnp.int32),
            pltpu.VMEM((BLK, HID), jnp.float32),
            pltpu.SemaphoreType.DMA,
        ],
    )
    def k(e2_hbm, i_hbm, z_hbm, out_hbm, acc, idx_v, rows_v, sem):
        core = lax.axis_index("c")
        sid = lax.axis_index("s")
        pltpu.sync_copy(z_hbm, acc.at[pl.ds(sid * stripe, stripe)])
        plsc.subcore_barrier()
        base = (core * NSUB + sid) * per_tile

        def body(j, _):
            off = base + j * BLK
            c1 = pltpu.async_copy(i_hbm.at[pl.ds(off, BLK)], idx_v, sem)
            c2 = pltpu.async_copy(e2_hbm.at[pl.ds(off, BLK)], rows_v, sem)
            c1.wait()
            c2.wait()
            pltpu.sync_copy(rows_v, acc.at[idx_v], add=True)
            return 0

        lax.fori_loop(0, nblk, body, 0)
        plsc.subcore_barrier()
        pltpu.sync_copy(acc.at[pl.ds(sid * stripe, stripe)],
                        out_hbm.at[core, pl.ds(sid * stripe, stripe)])

    return k(e2, i_idx, zrows)


def _stage_a_body(e1_ref, rbf_ref, wji_ref, bji_ref, wkj_ref, bkj_ref,
                  r1_ref, r2_ref, xji_ref, zkj_ref):
    e1 = e1_ref[...]
    xji_ref[...] = _swish(_mm(e1, wji_ref[...]) + bji_ref[...])
    xkj = _swish(_mm(e1, wkj_ref[...]) + bkj_ref[...])
    rbf = (rbf_ref[...] @ r1_ref[...]) @ r2_ref[...]
    zkj_ref[...] = xkj * rbf


def _stage_a(e1, rbf0, p):
    E = e1.shape[0]
    return pl.pallas_call(
        _stage_a_body,
        grid=(E // BE,),
        in_specs=[
            pl.BlockSpec((BE, HID), lambda i: (i, 0)),
            pl.BlockSpec((BE, NRAD), lambda i: (i, 0)),
            _full((HID, HID)), _full((1, HID)),
            _full((HID, HID)), _full((1, HID)),
            _full((NRAD, BASIS)), _full((BASIS, HID)),
        ],
        out_specs=[
            pl.BlockSpec((BE, HID), lambda i: (i, 0)),
            pl.BlockSpec((BE, HID), lambda i: (i, 0)),
        ],
        out_shape=[
            jax.ShapeDtypeStruct((E, HID), jnp.float32),
            jax.ShapeDtypeStruct((E, HID), jnp.float32),
        ],
        interpret=_INTERPRET,
    )(e1, rbf0, p['lin_ji_w'], p['lin_ji_b'][None, :],
      p['lin_kj_w'], p['lin_kj_b'][None, :],
      p['lin_rbf1'], p['lin_rbf2'])


def _stage_t_body(xg_ref, dkj_ref, ang_ref, wdn_ref, s1_ref, s2_ref, wup_ref,
                  y_ref):
    d = dkj_ref[...] / CUTOFF
    rbf = _envelope(d) * _sin_ladder(d)
    ca = jnp.cos(ang_ref[...])
    ts = [jnp.ones_like(ca), ca]
    for _ in range(NSPH - 2):
        ts.append(2.0 * ca * ts[-1] - ts[-2])
    sb8 = ts[0] * (rbf @ s1_ref[0])
    for s in range(1, NSPH):
        sb8 = sb8 + ts[s] * (rbf @ s1_ref[s])
    xdn = _swish(_mm(xg_ref[...], wdn_ref[...]))
    y_ref[...] = _mm(xdn * (sb8 @ s2_ref[...]), wup_ref[...])


def _stage_t(xg, dist_kj, angle, p):
    T = xg.shape[0]
    s1 = p['lin_sbf1'].reshape(NSPH, NRAD, BASIS)
    return pl.pallas_call(
        _stage_t_body,
        grid=(T // BT,),
        in_specs=[
            pl.BlockSpec((BT, HID), lambda i: (i, 0)),
            pl.BlockSpec((BT, 1), lambda i: (i, 0)),
            pl.BlockSpec((BT, 1), lambda i: (i, 0)),
            _full((HID, INT_EMB)),
            _full((NSPH, NRAD, BASIS)),
            _full((BASIS, INT_EMB)),
            _full((INT_EMB, HID)),
        ],
        out_specs=pl.BlockSpec((BT, HID), lambda i: (i, 0)),
        out_shape=jax.ShapeDtypeStruct((T, HID), jnp.float32),
        interpret=_INTERPRET,
    )(xg, dist_kj[:, None], angle[:, None], p['lin_down'], s1, p['lin_sbf2'],
      p['lin_up'])


def _stage_s_body(xji_ref, xsa_ref, xsb_ref, rbf_ref, e1_ref,
                  wcat, bw1, bb1, bw2, bb2, lw, lb,
                  a1w1, a1b1, a1w2, a1b2, a2w1, a2b1, a2w2, a2b2, lrbf,
                  e1o_ref, e2o_ref):
    xkj = _swish(xsa_ref[0] + xsb_ref[0])
    e = _swish(_mm(xji_ref[...] + xkj, wcat[...]))
    e = e + _swish(_mm(_swish(_mm(e, bw1[...]) + bb1[...]), bw2[...]) + bb2[...])
    e = _swish(_mm(e, lw[...]) + lb[...]) + e1_ref[...]
    e = e + _swish(_mm(_swish(_mm(e, a1w1[...]) + a1b1[...]), a1w2[...]) + a1b2[...])
    e = e + _swish(_mm(_swish(_mm(e, a2w1[...]) + a2b1[...]), a2w2[...]) + a2b2[...])
    e1o_ref[...] = e
    e2o_ref[...] = (rbf_ref[...] @ lrbf[...]) * e


def _stage_s(xji, xs, rbf0, e1, p):
    E = xji.shape[0]
    b = p['before'][0]
    a1, a2 = p['after']
    return pl.pallas_call(
        _stage_s_body,
        grid=(E // BE,),
        in_specs=[
            pl.BlockSpec((BE, HID), lambda i: (i, 0)),
            pl.BlockSpec((1, BE, HID), lambda i: (0, i, 0)),
            pl.BlockSpec((1, BE, HID), lambda i: (1, i, 0)),
            pl.BlockSpec((BE, NRAD), lambda i: (i, 0)),
            pl.BlockSpec((BE, HID), lambda i: (i, 0)),
            _full((HID, HID)),
            _full((HID, HID)), _full((1, HID)), _full((HID, HID)), _full((1, HID)),
            _full((HID, HID)), _full((1, HID)),
            _full((HID, HID)), _full((1, HID)), _full((HID, HID)), _full((1, HID)),
            _full((HID, HID)), _full((1, HID)), _full((HID, HID)), _full((1, HID)),
            _full((NRAD, HID)),
        ],
        out_specs=[
            pl.BlockSpec((BE, HID), lambda i: (i, 0)),
            pl.BlockSpec((BE, HID), lambda i: (i, 0)),
        ],
        out_shape=[
            jax.ShapeDtypeStruct((E, HID), jnp.float32),
            jax.ShapeDtypeStruct((E, HID), jnp.float32),
        ],
        interpret=_INTERPRET,
    )(xji, xs, xs, rbf0, e1,
      p['lin_cat'],
      b['w1'], b['b1'][None, :], b['w2'], b['b2'][None, :],
      p['lin_w'], p['lin_b'][None, :],
      a1['w1'], a1['b1'][None, :], a1['w2'], a1['b2'][None, :],
      a2['w1'], a2['b1'][None, :], a2['w2'], a2['b2'][None, :],
      p['lin_rbf'])


def _upd_v_body(va_ref, vb_ref, batch_ref, upw, upb, l1w, l1b, l2w, l2b,
                l3w, l3b, outw, u_ref):
    v = _mm(va_ref[0] + vb_ref[0], upw[...]) + upb[...]
    v = _swish(_mm(v, l1w[...]) + l1b[...])
    v = _swish(_mm(v, l2w[...]) + l2b[...])
    v = _swish(_mm(v, l3w[...]) + l3b[...])
    vv = v @ outw[...]
    gid = lax.broadcasted_iota(jnp.int32, (N_G, vv.shape[0]), 0)
    onehot = (gid == batch_ref[0]).astype(jnp.float32)

    @pl.when(pl.program_id(0) == 0)
    def _():
        u_ref[...] = jnp.zeros_like(u_ref)

    u_ref[...] += onehot @ vv


def _upd_v(na, batch_pad3, q):
    ls = q['lins']
    return pl.pallas_call(
        _upd_v_body,
        grid=(NPAD // BNV,),
        in_specs=[
            pl.BlockSpec((1, BNV, HID), lambda i: (0, i, 0)),
            pl.BlockSpec((1, BNV, HID), lambda i: (1, i, 0)),
            pl.BlockSpec((1, 1, BNV), lambda i: (i, 0, 0)),
            _full((HID, OUT_EMB)), _full((1, OUT_EMB)),
            _full((OUT_EMB, OUT_EMB)), _full((1, OUT_EMB)),
            _full((OUT_EMB, OUT_EMB)), _full((1, OUT_EMB)),
            _full((OUT_EMB, OUT_EMB)), _full((1, OUT_EMB)),
            _full((OUT_EMB, 1)),
        ],
        out_specs=_full((N_G, 1)),
        out_shape=jax.ShapeDtypeStruct((N_G, 1), jnp.float32),
        interpret=_INTERPRET,
    )(na, na, batch_pad3, q['up_w'], q['up_b'][None, :],
      ls[0]['w'], ls[0]['b'][None, :],
      ls[1]['w'], ls[1]['b'][None, :],
      ls[2]['w'], ls[2]['b'][None, :],
      q['out_w'])


def kernel(x, edge_attr, dist, angle, params, idx_kj, idx_ji, i, batch):
    N = x.shape[0]
    E = edge_attr.shape[0]
    T = angle.shape[0]

    sorted_ji, sorted_kj, angle_st = lax.sort((idx_ji, idx_kj, angle),
                                              num_keys=1)
    starts = jnp.searchsorted(
        sorted_ji, jnp.arange(NCHUNK + 1, dtype=jnp.int32) * WIN).astype(jnp.int32)
    s_al = starts[:-1] & ~7
    e_raw = starts[1:]
    bounds = jnp.stack(
        [s_al, e_raw] + [jnp.zeros((NCHUNK,), jnp.int32)] * 14, axis=1)
    pad = TPAD - T
    idx_kj2d = jnp.concatenate(
        [sorted_kj, jnp.zeros((pad,), jnp.int32)]).reshape(TPAD // 128, 128)
    idx_ji_pad = jnp.concatenate(
        [sorted_ji, jnp.full((pad,), E, jnp.int32)])
    dist_kj_s = jnp.concatenate(
        [jnp.take(dist, sorted_kj), jnp.ones((pad,), jnp.float32)])
    angle_s = jnp.concatenate([angle_st, jnp.zeros((pad,), jnp.float32)])
    batch_pad3 = jnp.concatenate(
        [batch, jnp.full((NPAD - N,), N_G, jnp.int32)]).reshape(
            NPAD // BNV, 1, BNV)
    zrows_t = jnp.zeros((WACC // NSUB // 3, HID), jnp.float32)
    zrows_n = jnp.zeros((NPAD // NSUB, HID), jnp.float32)

    e1, rbf0 = _edge_init(edge_attr, dist, params['lin_edge'])
    u = _node_init(x, batch, params['lin_node'])

    for pe, pv in zip(params['update_es'], params['update_vs']):
        xji, zkj = _stage_a(e1, rbf0, pe)
        xg = _sc_gather(zkj, idx_kj2d)
        y = _stage_t(xg, dist_kj_s, angle_s, pe)
        xs = _sc_tri_scatter(y, idx_ji_pad, bounds, zrows_t)
        e1, e2 = _stage_s(xji, xs, rbf0, e1, pe)
        na = _sc_node_scatter(e2, i, zrows_n)
        u = u + _upd_v(na, batch_pad3, pv)
    return u


# speedup vs baseline: 1.0543x; 1.0543x over previous
import functools

import jax
import jax.numpy as jnp
import numpy as np
from jax import lax
from jax.experimental import pallas as pl
from jax.experimental.pallas import tpu as pltpu
from jax.experimental.pallas import tpu_sc as plsc

NC = 2
NSUB = 16
NW = NC * NSUB

CUTOFF = 5.0
NRAD = 6
NSPH = 7
ENV_P = 5
HID = 128
INT_EMB = 64
BASIS = 8
OUT_EMB = 256
N_G = 64

BE = 2000
BT = 4096
BN = 2000
BNV = 2048

TPAD = 655360
NPAD = 10240
WIN = 8000
NCHUNK = 40
WACC = 8064

_INTERPRET = False


def _swish(x):
    return x / (1.0 + jnp.exp(-x))


def _mm(a, b):
    return jnp.dot(a.astype(jnp.bfloat16), b.astype(jnp.bfloat16),
                   preferred_element_type=jnp.float32)


def _envelope(d):
    p = ENV_P
    a = -(p + 1) * (p + 2) / 2.0
    b = float(p * (p + 2))
    c = -p * (p + 1) / 2.0
    d2 = d * d
    d4 = d2 * d2
    d5 = d4 * d
    return 1.0 / d + a * d5 + b * d5 * d + c * d5 * d2


def _sin_ladder(d):
    s1 = jnp.sin(jnp.pi * d)
    c1 = jnp.cos(jnp.pi * d)
    sins = [s1]
    ck = c1
    for _ in range(NRAD - 1):
        sk = sins[-1]
        sins.append(sk * c1 + ck * s1)
        ck = ck * c1 - sk * s1
    return jnp.concatenate(sins, axis=1)


def _full(spec_shape):
    return pl.BlockSpec(spec_shape, lambda *_: tuple(0 for _ in spec_shape))


def _edge_init_body(ea_ref, dist_ref, we_ref, e1_ref, rbf_ref):
    e1_ref[...] = ea_ref[...] @ we_ref[...]
    d = dist_ref[...] / CUTOFF
    rbf_ref[...] = _envelope(d) * _sin_ladder(d)


def _edge_init(edge_attr, dist, we):
    E = edge_attr.shape[0]
    grid = (E // BE,)
    return pl.pallas_call(
        _edge_init_body,
        grid=grid,
        in_specs=[
            pl.BlockSpec((BE, 12), lambda i: (i, 0)),
            pl.BlockSpec((BE, 1), lambda i: (i, 0)),
            _full((12, HID)),
        ],
        out_specs=[
            pl.BlockSpec((BE, HID), lambda i: (i, 0)),
            pl.BlockSpec((BE, NRAD), lambda i: (i, 0)),
        ],
        out_shape=[
            jax.ShapeDtypeStruct((E, HID), jnp.float32),
            jax.ShapeDtypeStruct((E, NRAD), jnp.float32),
        ],
        interpret=_INTERPRET,
    )(edge_attr, dist[:, None], we)


def _node_init_body(x_ref, batch_ref, wn_ref, u_ref):
    v = x_ref[...] @ wn_ref[...]
    gid = lax.broadcasted_iota(jnp.int32, (N_G, v.shape[0]), 0)
    onehot = (gid == batch_ref[0]).astype(jnp.float32)

    @pl.when(pl.program_id(0) == 0)
    def _():
        u_ref[...] = jnp.zeros_like(u_ref)

    u_ref[...] += onehot @ v


def _node_init(x, batch, wn):
    N = x.shape[0]
    return pl.pallas_call(
        _node_init_body,
        grid=(N // BN,),
        in_specs=[
            pl.BlockSpec((BN, 48), lambda i: (i, 0)),
            pl.BlockSpec((1, 1, BN), lambda i: (i, 0, 0)),
            _full((48, HID)),
        ],
        out_specs=_full((N_G, HID)),
        out_shape=jax.ShapeDtypeStruct((N_G, HID), jnp.float32),
        interpret=_INTERPRET,
    )(x, batch.reshape(N // BN, 1, BN), wn)


_MESH = plsc.VectorSubcoreMesh(core_axis_name="c", subcore_axis_name="s",
                               num_cores=NC, num_subcores=NSUB)


def _sc_gather(xdn, idx_kj2d):
    per_tile = TPAD // NW
    ngrp = per_tile // (4 * 128)

    @functools.partial(
        pl.kernel,
        out_type=jax.ShapeDtypeStruct((TPAD, HID), jnp.float32),
        mesh=_MESH,
        scratch_types=[
            pltpu.VMEM((4, 128), jnp.int32),
            pltpu.VMEM((4, 128, HID), jnp.float32),
            pltpu.SemaphoreType.DMA, pltpu.SemaphoreType.DMA,
        ],
    )
    def k(xdn_hbm, kj_hbm, xg_hbm, idx_v, rows_v, gsem, ssem):
        wid = lax.axis_index("c") * NSUB + lax.axis_index("s")
        grp0 = wid * (ngrp * 4)

        def body(j, _):
            blk = grp0 + j * 4
            pltpu.sync_copy(kj_hbm.at[pl.ds(blk, 4)], idx_v)

            @pl.when(j > 0)
            def _():
                for b in range(4):
                    pltpu.make_async_copy(
                        xg_hbm.at[pl.ds(0, 128)], rows_v.at[b], ssem).wait()

            descs = [pltpu.async_copy(xdn_hbm.at[idx_v.at[b]], rows_v.at[b],
                                      gsem) for b in range(4)]
            for d in descs:
                d.wait()
            for b in range(4):
                pltpu.async_copy(rows_v.at[b],
                                 xg_hbm.at[pl.ds((blk + b) * 128, 128)], ssem)
            return 0

        lax.fori_loop(0, ngrp, body, 0)
        for b in range(4):
            pltpu.make_async_copy(xg_hbm.at[pl.ds(0, 128)], rows_v.at[b],
                                  ssem).wait()

    return k(xdn, idx_kj2d)


def _sc_tri_scatter(y, idx_ji_pad, bounds, zrows):
    zstripe = WACC // NSUB

    @functools.partial(
        pl.kernel,
        out_type=jax.ShapeDtypeStruct((NC, 320000 + 64, HID), jnp.float32),
        mesh=_MESH,
        scratch_types=[
            pltpu.VMEM_SHARED((WACC, HID), jnp.float32),
            pltpu.VMEM((zstripe // 3, HID), jnp.float32),
            pltpu.VMEM((2, 128), jnp.int32),
            pltpu.VMEM((2, 128), jnp.int32),
            pltpu.VMEM((2, 128, HID), jnp.float32),
            pltpu.VMEM((NCHUNK, 16), jnp.int32),
            pltpu.SemaphoreType.DMA,
            pltpu.SemaphoreType.DMA,
        ],
    )
    def k(y_hbm, ji_hbm, bounds_hbm, z_hbm, out_hbm,
          acc, zbuf, idxj_v, idxc_v, rows_v, bounds_v, lsem, ssem):
        core = lax.axis_index("c")
        sid = lax.axis_index("s")
        wid = core * NSUB + sid
        pltpu.sync_copy(z_hbm, zbuf)
        pltpu.sync_copy(bounds_hbm, bounds_v)

        for c in range(NCHUNK):
            for z in range(3):
                pltpu.sync_copy(
                    zbuf,
                    acc.at[pl.ds(sid * zstripe + z * (zstripe // 3),
                                 zstripe // 3)])
            plsc.subcore_barrier()
            row = bounds_v[c]
            s = row[0]
            e = row[1]
            nb = lax.shift_right_logical(e - s + (NW * 128 - 1), 12)
            nb2 = lax.shift_right_logical(nb + 1, 1)
            base = c * WIN

            def inner(g, _, s=s, base=base):
                offs = []
                loads = []
                for p in range(2):
                    off = pl.multiple_of(
                        s + ((2 * g + p) * NW + wid) * 128, 8)
                    offs.append(off)
                    loads.append(pltpu.async_copy(
                        ji_hbm.at[pl.ds(off, 128)], idxj_v.at[p], lsem))
                    loads.append(pltpu.async_copy(
                        y_hbm.at[pl.ds(off, 128)], rows_v.at[p], lsem))
                scats = []
                for p in range(2):
                    loads[2 * p].wait()
                    loads[2 * p + 1].wait()
                    for m in range(8):
                        lv = idxj_v[p, pl.ds(m * 16, 16)] - base
                        inb = (lv >= 0) & (lv < WIN)
                        idxc_v[p, pl.ds(m * 16, 16)] = jnp.where(
                            inb, lv, WIN)
                    scats.append(pltpu.async_copy(
                        rows_v.at[p], acc.at[idxc_v.at[p]], ssem, add=True))
                for sc in scats:
                    sc.wait()
                return 0

            lax.fori_loop(0, nb2, inner, 0)
            plsc.subcore_barrier()
            pltpu.sync_copy(
                acc.at[pl.ds(sid * zstripe, zstripe)],
                out_hbm.at[core, pl.ds(base + sid * zstripe, zstripe)])
            plsc.subcore_barrier()

    return k(y, idx_ji_pad, bounds, zrows)


def _sc_node_scatter(e2, i_idx, zrows):
    E = e2.shape[0]
    per_core = E // NC
    per_tile = per_core // NSUB
    BLK = 80
    nblk = per_tile // BLK
    stripe = NPAD // NSUB

    @functools.partial(
        pl.kernel,
        out_type=jax.ShapeDtypeStruct((NC, NPAD, HID), jnp.float32),
        mesh=_MESH,
        scratch_types=[
            pltpu.VMEM_SHARED((NPAD, HID), jnp.float32),
            pltpu.VMEM((2, BLK), jnp.int32),
            pltpu.VMEM((2, BLK, HID), jnp.float32),
            pltpu.SemaphoreType.DMA,
            pltpu.SemaphoreType.DMA,
        ],
    )
    def k(e2_hbm, i_hbm, z_hbm, out_hbm, acc, idx_v, rows_v, lsem, ssem):
        core = lax.axis_index("c")
        sid = lax.axis_index("s")
        pltpu.sync_copy(z_hbm, acc.at[pl.ds(sid * stripe, stripe)])
        plsc.subcore_barrier()
        base = (core * NSUB + sid) * per_tile

        def pair(g, _):
            loads = []
            for p in range(2):
                off = base + (2 * g + p) * BLK
                loads.append(pltpu.async_copy(
                    i_hbm.at[pl.ds(off, BLK)], idx_v.at[p], lsem))
                loads.append(pltpu.async_copy(
                    e2_hbm.at[pl.ds(off, BLK)], rows_v.at[p], lsem))
            scats = []
            for p in range(2):
                loads[2 * p].wait()
                loads[2 * p + 1].wait()
                scats.append(pltpu.async_copy(
                    rows_v.at[p], acc.at[idx_v.at[p]], ssem, add=True))
            for sc in scats:
                sc.wait()
            return 0

        lax.fori_loop(0, nblk // 2, pair, 0)
        off = base + (nblk - 1) * BLK
        pltpu.sync_copy(i_hbm.at[pl.ds(off, BLK)], idx_v.at[0])
        pltpu.sync_copy(e2_hbm.at[pl.ds(off, BLK)], rows_v.at[0])
        pltpu.sync_copy(rows_v.at[0], acc.at[idx_v.at[0]], add=True)
        plsc.subcore_barrier()
        pltpu.sync_copy(acc.at[pl.ds(sid * stripe, stripe)],
                        out_hbm.at[core, pl.ds(sid * stripe, stripe)])

    return k(e2, i_idx, zrows)


def _stage_a_body(e1_ref, rbf_ref, wkj_ref, bkj_ref,
                  r1_ref, r2_ref, zkj_ref):
    e1 = e1_ref[...]
    xkj = _swish(_mm(e1, wkj_ref[...]) + bkj_ref[...])
    rbf = (rbf_ref[...] @ r1_ref[...]) @ r2_ref[...]
    zkj_ref[...] = xkj * rbf


def _stage_a(e1, rbf0, p):
    E = e1.shape[0]
    return pl.pallas_call(
        _stage_a_body,
        grid=(E // BE,),
        in_specs=[
            pl.BlockSpec((BE, HID), lambda i: (i, 0)),
            pl.BlockSpec((BE, NRAD), lambda i: (i, 0)),
            _full((HID, HID)), _full((1, HID)),
            _full((NRAD, BASIS)), _full((BASIS, HID)),
        ],
        out_specs=pl.BlockSpec((BE, HID), lambda i: (i, 0)),
        out_shape=jax.ShapeDtypeStruct((E, HID), jnp.float32),
        interpret=_INTERPRET,
    )(e1, rbf0,
      p['lin_kj_w'], p['lin_kj_b'][None, :],
      p['lin_rbf1'], p['lin_rbf2'])


def _stage_t_body(xg_ref, dkj_ref, ang_ref, wdn_ref, s1_ref, s2_ref, wup_ref,
                  y_ref):
    d = dkj_ref[...] / CUTOFF
    rbf = _envelope(d) * _sin_ladder(d)
    ca = jnp.cos(ang_ref[...])
    ts = [jnp.ones_like(ca), ca]
    for _ in range(NSPH - 2):
        ts.append(2.0 * ca * ts[-1] - ts[-2])
    sb8 = ts[0] * (rbf @ s1_ref[0])
    for s in range(1, NSPH):
        sb8 = sb8 + ts[s] * (rbf @ s1_ref[s])
    xdn = _swish(_mm(xg_ref[...], wdn_ref[...]))
    y_ref[...] = _mm(xdn * (sb8 @ s2_ref[...]), wup_ref[...])


def _stage_t(xg, dist_kj, angle, p):
    T = xg.shape[0]
    s1 = p['lin_sbf1'].reshape(NSPH, NRAD, BASIS)
    return pl.pallas_call(
        _stage_t_body,
        grid=(T // BT,),
        in_specs=[
            pl.BlockSpec((BT, HID), lambda i: (i, 0)),
            pl.BlockSpec((BT, 1), lambda i: (i, 0)),
            pl.BlockSpec((BT, 1), lambda i: (i, 0)),
            _full((HID, INT_EMB)),
            _full((NSPH, NRAD, BASIS)),
            _full((BASIS, INT_EMB)),
            _full((INT_EMB, HID)),
        ],
        out_specs=pl.BlockSpec((BT, HID), lambda i: (i, 0)),
        out_shape=jax.ShapeDtypeStruct((T, HID), jnp.float32),
        interpret=_INTERPRET,
    )(xg, dist_kj[:, None], angle[:, None], p['lin_down'], s1, p['lin_sbf2'],
      p['lin_up'])


def _stage_s_body(xsa_ref, xsb_ref, rbf_ref, e1_ref,
                  wji, bji, wcat, bw1, bb1, bw2, bb2, lw, lb,
                  a1w1, a1b1, a1w2, a1b2, a2w1, a2b1, a2w2, a2b2, lrbf,
                  e1o_ref, e2o_ref):
    e1 = e1_ref[...]
    xji = _swish(_mm(e1, wji[...]) + bji[...])
    xkj = _swish(xsa_ref[0] + xsb_ref[0])
    e = _swish(_mm(xji + xkj, wcat[...]))
    e = e + _swish(_mm(_swish(_mm(e, bw1[...]) + bb1[...]), bw2[...]) + bb2[...])
    e = _swish(_mm(e, lw[...]) + lb[...]) + e1
    e = e + _swish(_mm(_swish(_mm(e, a1w1[...]) + a1b1[...]), a1w2[...]) + a1b2[...])
    e = e + _swish(_mm(_swish(_mm(e, a2w1[...]) + a2b1[...]), a2w2[...]) + a2b2[...])
    e1o_ref[...] = e
    e2o_ref[...] = (rbf_ref[...] @ lrbf[...]) * e


def _stage_s(xs, rbf0, e1, p):
    E = e1.shape[0]
    b = p['before'][0]
    a1, a2 = p['after']
    return pl.pallas_call(
        _stage_s_body,
        grid=(E // BE,),
        in_specs=[
            pl.BlockSpec((1, BE, HID), lambda i: (0, i, 0)),
            pl.BlockSpec((1, BE, HID), lambda i: (1, i, 0)),
            pl.BlockSpec((BE, NRAD), lambda i: (i, 0)),
            pl.BlockSpec((BE, HID), lambda i: (i, 0)),
            _full((HID, HID)), _full((1, HID)), _full((HID, HID)),
            _full((HID, HID)), _full((1, HID)), _full((HID, HID)), _full((1, HID)),
            _full((HID, HID)), _full((1, HID)),
            _full((HID, HID)), _full((1, HID)), _full((HID, HID)), _full((1, HID)),
            _full((HID, HID)), _full((1, HID)), _full((HID, HID)), _full((1, HID)),
            _full((NRAD, HID)),
        ],
        out_specs=[
            pl.BlockSpec((BE, HID), lambda i: (i, 0)),
            pl.BlockSpec((BE, HID), lambda i: (i, 0)),
        ],
        out_shape=[
            jax.ShapeDtypeStruct((E, HID), jnp.float32),
            jax.ShapeDtypeStruct((E, HID), jnp.float32),
        ],
        interpret=_INTERPRET,
    )(xs, xs, rbf0, e1,
      p['lin_ji_w'], p['lin_ji_b'][None, :], p['lin_cat'],
      b['w1'], b['b1'][None, :], b['w2'], b['b2'][None, :],
      p['lin_w'], p['lin_b'][None, :],
      a1['w1'], a1['b1'][None, :], a1['w2'], a1['b2'][None, :],
      a2['w1'], a2['b1'][None, :], a2['w2'], a2['b2'][None, :],
      p['lin_rbf'])


def _upd_v_body(va_ref, vb_ref, batch_ref, upw, upb, l1w, l1b, l2w, l2b,
                l3w, l3b, outw, u_ref):
    v = _mm(va_ref[0] + vb_ref[0], upw[...]) + upb[...]
    v = _swish(_mm(v, l1w[...]) + l1b[...])
    v = _swish(_mm(v, l2w[...]) + l2b[...])
    v = _swish(_mm(v, l3w[...]) + l3b[...])
    vv = v @ outw[...]
    gid = lax.broadcasted_iota(jnp.int32, (N_G, vv.shape[0]), 0)
    onehot = (gid == batch_ref[0]).astype(jnp.float32)

    @pl.when(pl.program_id(0) == 0)
    def _():
        u_ref[...] = jnp.zeros_like(u_ref)

    u_ref[...] += onehot @ vv


def _upd_v(na, batch_pad3, q):
    ls = q['lins']
    return pl.pallas_call(
        _upd_v_body,
        grid=(NPAD // BNV,),
        in_specs=[
            pl.BlockSpec((1, BNV, HID), lambda i: (0, i, 0)),
            pl.BlockSpec((1, BNV, HID), lambda i: (1, i, 0)),
            pl.BlockSpec((1, 1, BNV), lambda i: (i, 0, 0)),
            _full((HID, OUT_EMB)), _full((1, OUT_EMB)),
            _full((OUT_EMB, OUT_EMB)), _full((1, OUT_EMB)),
            _full((OUT_EMB, OUT_EMB)), _full((1, OUT_EMB)),
            _full((OUT_EMB, OUT_EMB)), _full((1, OUT_EMB)),
            _full((OUT_EMB, 1)),
        ],
        out_specs=_full((N_G, 1)),
        out_shape=jax.ShapeDtypeStruct((N_G, 1), jnp.float32),
        interpret=_INTERPRET,
    )(na, na, batch_pad3, q['up_w'], q['up_b'][None, :],
      ls[0]['w'], ls[0]['b'][None, :],
      ls[1]['w'], ls[1]['b'][None, :],
      ls[2]['w'], ls[2]['b'][None, :],
      q['out_w'])


def kernel(x, edge_attr, dist, angle, params, idx_kj, idx_ji, i, batch):
    N = x.shape[0]
    E = edge_attr.shape[0]
    T = angle.shape[0]

    sorted_ji, sorted_kj, angle_st = lax.sort((idx_ji, idx_kj, angle),
                                              num_keys=1)
    starts = jnp.searchsorted(
        sorted_ji, jnp.arange(NCHUNK + 1, dtype=jnp.int32) * WIN).astype(jnp.int32)
    s_al = starts[:-1] & ~7
    e_raw = starts[1:]
    bounds = jnp.stack(
        [s_al, e_raw] + [jnp.zeros((NCHUNK,), jnp.int32)] * 14, axis=1)
    pad = TPAD - T
    idx_kj2d = jnp.concatenate(
        [sorted_kj, jnp.zeros((pad,), jnp.int32)]).reshape(TPAD // 128, 128)
    idx_ji_pad = jnp.concatenate(
        [sorted_ji, jnp.full((pad,), E, jnp.int32)])
    dist_kj_s = jnp.concatenate(
        [jnp.take(dist, sorted_kj), jnp.ones((pad,), jnp.float32)])
    angle_s = jnp.concatenate([angle_st, jnp.zeros((pad,), jnp.float32)])
    batch_pad3 = jnp.concatenate(
        [batch, jnp.full((NPAD - N,), N_G, jnp.int32)]).reshape(
            NPAD // BNV, 1, BNV)
    zrows_t = jnp.zeros((WACC // NSUB // 3, HID), jnp.float32)
    zrows_n = jnp.zeros((NPAD // NSUB, HID), jnp.float32)

    e1, rbf0 = _edge_init(edge_attr, dist, params['lin_edge'])
    u = _node_init(x, batch, params['lin_node'])

    for pe, pv in zip(params['update_es'], params['update_vs']):
        zkj = _stage_a(e1, rbf0, pe)
        xg = _sc_gather(zkj, idx_kj2d)
        y = _stage_t(xg, dist_kj_s, angle_s, pe)
        xs = _sc_tri_scatter(y, idx_ji_pad, bounds, zrows_t)
        e1, e2 = _stage_s(xs, rbf0, e1, pe)
        na = _sc_node_scatter(e2, i, zrows_n)
        u = u + _upd_v(na, batch_pad3, pv)
    return u


# speedup vs baseline: 1.1559x; 1.0964x over previous
import functools

import jax
import jax.numpy as jnp
import numpy as np
from jax import lax
from jax.experimental import pallas as pl
from jax.experimental.pallas import tpu as pltpu
from jax.experimental.pallas import tpu_sc as plsc

NC = 2
NSUB = 16
NW = NC * NSUB

CUTOFF = 5.0
NRAD = 6
NSPH = 7
ENV_P = 5
HID = 128
INT_EMB = 64
BASIS = 8
OUT_EMB = 256
N_G = 64

BE = 2000
BT = 4096
BN = 2000
BNV = 2048

TPAD = 655360
NPAD = 10240
WIN = 8000
NCHUNK = 40
WACC = 8064

_INTERPRET = False


def _swish(x):
    return x / (1.0 + jnp.exp(-x))


def _mm(a, b):
    return jnp.dot(a.astype(jnp.bfloat16), b.astype(jnp.bfloat16),
                   preferred_element_type=jnp.float32)


def _envelope(d):
    p = ENV_P
    a = -(p + 1) * (p + 2) / 2.0
    b = float(p * (p + 2))
    c = -p * (p + 1) / 2.0
    d2 = d * d
    d4 = d2 * d2
    d5 = d4 * d
    return 1.0 / d + a * d5 + b * d5 * d + c * d5 * d2


def _sin_ladder(d):
    s1 = jnp.sin(jnp.pi * d)
    c1 = jnp.cos(jnp.pi * d)
    sins = [s1]
    ck = c1
    for _ in range(NRAD - 1):
        sk = sins[-1]
        sins.append(sk * c1 + ck * s1)
        ck = ck * c1 - sk * s1
    return jnp.concatenate(sins, axis=1)


def _full(spec_shape):
    return pl.BlockSpec(spec_shape, lambda *_: tuple(0 for _ in spec_shape))


def _edge_init_body(ea_ref, dist_ref, we_ref, e1_ref, rbf_ref):
    e1_ref[...] = ea_ref[...] @ we_ref[...]
    d = dist_ref[...] / CUTOFF
    rbf_ref[...] = _envelope(d) * _sin_ladder(d)


def _edge_init(edge_attr, dist, we):
    E = edge_attr.shape[0]
    grid = (E // BE,)
    return pl.pallas_call(
        _edge_init_body,
        grid=grid,
        in_specs=[
            pl.BlockSpec((BE, 12), lambda i: (i, 0)),
            pl.BlockSpec((BE, 1), lambda i: (i, 0)),
            _full((12, HID)),
        ],
        out_specs=[
            pl.BlockSpec((BE, HID), lambda i: (i, 0)),
            pl.BlockSpec((BE, NRAD), lambda i: (i, 0)),
        ],
        out_shape=[
            jax.ShapeDtypeStruct((E, HID), jnp.float32),
            jax.ShapeDtypeStruct((E, NRAD), jnp.float32),
        ],
        interpret=_INTERPRET,
    )(edge_attr, dist[:, None], we)


def _node_init_body(x_ref, batch_ref, wn_ref, u_ref):
    v = x_ref[...] @ wn_ref[...]
    gid = lax.broadcasted_iota(jnp.int32, (N_G, v.shape[0]), 0)
    onehot = (gid == batch_ref[0]).astype(jnp.float32)

    @pl.when(pl.program_id(0) == 0)
    def _():
        u_ref[...] = jnp.zeros_like(u_ref)

    u_ref[...] += onehot @ v


def _node_init(x, batch, wn):
    N = x.shape[0]
    return pl.pallas_call(
        _node_init_body,
        grid=(N // BN,),
        in_specs=[
            pl.BlockSpec((BN, 48), lambda i: (i, 0)),
            pl.BlockSpec((1, 1, BN), lambda i: (i, 0, 0)),
            _full((48, HID)),
        ],
        out_specs=_full((N_G, HID)),
        out_shape=jax.ShapeDtypeStruct((N_G, HID), jnp.float32),
        interpret=_INTERPRET,
    )(x, batch.reshape(N // BN, 1, BN), wn)


_MESH = plsc.VectorSubcoreMesh(core_axis_name="c", subcore_axis_name="s",
                               num_cores=NC, num_subcores=NSUB)


def _sc_gather(xdn, idx_kj2d):
    per_tile = TPAD // NW
    ngrp = per_tile // (4 * 128)

    @functools.partial(
        pl.kernel,
        out_type=jax.ShapeDtypeStruct((TPAD, HID), jnp.float32),
        mesh=_MESH,
        scratch_types=[
            pltpu.VMEM((4, 128), jnp.int32),
            pltpu.VMEM((4, 128, HID), jnp.float32),
            pltpu.SemaphoreType.DMA, pltpu.SemaphoreType.DMA,
        ],
    )
    def k(xdn_hbm, kj_hbm, xg_hbm, idx_v, rows_v, gsem, ssem):
        wid = lax.axis_index("c") * NSUB + lax.axis_index("s")
        grp0 = wid * (ngrp * 4)

        def body(j, _):
            blk = grp0 + j * 4
            pltpu.sync_copy(kj_hbm.at[pl.ds(blk, 4)], idx_v)

            @pl.when(j > 0)
            def _():
                for b in range(4):
                    pltpu.make_async_copy(
                        xg_hbm.at[pl.ds(0, 128)], rows_v.at[b], ssem).wait()

            descs = [pltpu.async_copy(xdn_hbm.at[idx_v.at[b]], rows_v.at[b],
                                      gsem) for b in range(4)]
            for d in descs:
                d.wait()
            for b in range(4):
                pltpu.async_copy(rows_v.at[b],
                                 xg_hbm.at[pl.ds((blk + b) * 128, 128)], ssem)
            return 0

        lax.fori_loop(0, ngrp, body, 0)
        for b in range(4):
            pltpu.make_async_copy(xg_hbm.at[pl.ds(0, 128)], rows_v.at[b],
                                  ssem).wait()

    return k(xdn, idx_kj2d)


def _sc_tri_scatter(y, idx_ji_pad, bounds, zrows):
    zstripe = WACC // NSUB

    @functools.partial(
        pl.kernel,
        out_type=jax.ShapeDtypeStruct((NC, 320000 + 64, HID), jnp.float32),
        mesh=_MESH,
        scratch_types=[
            pltpu.VMEM_SHARED((WACC, HID), jnp.float32),
            pltpu.VMEM((zstripe // 3, HID), jnp.float32),
            pltpu.VMEM((2, 128), jnp.int32),
            pltpu.VMEM((2, 128), jnp.int32),
            pltpu.VMEM((2, 128, HID), jnp.float32),
            pltpu.VMEM((NCHUNK, 16), jnp.int32),
            pltpu.SemaphoreType.DMA,
            pltpu.SemaphoreType.DMA,
        ],
    )
    def k(y_hbm, ji_hbm, bounds_hbm, z_hbm, out_hbm,
          acc, zbuf, idxj_v, idxc_v, rows_v, bounds_v, lsem, ssem):
        core = lax.axis_index("c")
        sid = lax.axis_index("s")
        wid = core * NSUB + sid
        pltpu.sync_copy(z_hbm, zbuf)
        pltpu.sync_copy(bounds_hbm, bounds_v)

        for c in range(NCHUNK):
            for z in range(3):
                pltpu.sync_copy(
                    zbuf,
                    acc.at[pl.ds(sid * zstripe + z * (zstripe // 3),
                                 zstripe // 3)])
            plsc.subcore_barrier()
            row = bounds_v[c]
            s = row[0]
            e = row[1]
            nb = lax.shift_right_logical(e - s + (NW * 128 - 1), 12)
            nb2 = lax.shift_right_logical(nb + 1, 1)
            base = c * WIN

            def inner(g, _, s=s, base=base):
                offs = []
                loads = []
                for p in range(2):
                    off = pl.multiple_of(
                        s + ((2 * g + p) * NW + wid) * 128, 8)
                    offs.append(off)
                    loads.append(pltpu.async_copy(
                        ji_hbm.at[pl.ds(off, 128)], idxj_v.at[p], lsem))
                    loads.append(pltpu.async_copy(
                        y_hbm.at[pl.ds(off, 128)], rows_v.at[p], lsem))
                scats = []
                for p in range(2):
                    loads[2 * p].wait()
                    loads[2 * p + 1].wait()
                    for m in range(8):
                        lv = idxj_v[p, pl.ds(m * 16, 16)] - base
                        inb = (lv >= 0) & (lv < WIN)
                        idxc_v[p, pl.ds(m * 16, 16)] = jnp.where(
                            inb, lv, WIN)
                    scats.append(pltpu.async_copy(
                        rows_v.at[p], acc.at[idxc_v.at[p]], ssem, add=True))
                for sc in scats:
                    sc.wait()
                return 0

            lax.fori_loop(0, nb2, inner, 0)
            plsc.subcore_barrier()
            pltpu.sync_copy(
                acc.at[pl.ds(sid * zstripe, zstripe)],
                out_hbm.at[core, pl.ds(base + sid * zstripe, zstripe)])
            plsc.subcore_barrier()

    return k(y, idx_ji_pad, bounds, zrows)


def _sc_node_scatter(e2, i_idx, zrows):
    E = e2.shape[0]
    per_core = E // NC
    per_tile = per_core // NSUB
    BLK = 80
    nblk = per_tile // BLK
    stripe = NPAD // NSUB

    @functools.partial(
        pl.kernel,
        out_type=jax.ShapeDtypeStruct((NC, NPAD, HID), jnp.float32),
        mesh=_MESH,
        scratch_types=[
            pltpu.VMEM_SHARED((NPAD, HID), jnp.float32),
            pltpu.VMEM((2, BLK), jnp.int32),
            pltpu.VMEM((2, BLK, HID), jnp.float32),
            pltpu.SemaphoreType.DMA,
            pltpu.SemaphoreType.DMA,
        ],
    )
    def k(e2_hbm, i_hbm, z_hbm, out_hbm, acc, idx_v, rows_v, lsem, ssem):
        core = lax.axis_index("c")
        sid = lax.axis_index("s")
        pltpu.sync_copy(z_hbm, acc.at[pl.ds(sid * stripe, stripe)])
        plsc.subcore_barrier()
        base = (core * NSUB + sid) * per_tile

        def pair(g, _):
            loads = []
            for p in range(2):
                off = base + (2 * g + p) * BLK
                loads.append(pltpu.async_copy(
                    i_hbm.at[pl.ds(off, BLK)], idx_v.at[p], lsem))
                loads.append(pltpu.async_copy(
                    e2_hbm.at[pl.ds(off, BLK)], rows_v.at[p], lsem))
            scats = []
            for p in range(2):
                loads[2 * p].wait()
                loads[2 * p + 1].wait()
                scats.append(pltpu.async_copy(
                    rows_v.at[p], acc.at[idx_v.at[p]], ssem, add=True))
            for sc in scats:
                sc.wait()
            return 0

        lax.fori_loop(0, nblk // 2, pair, 0)
        off = base + (nblk - 1) * BLK
        pltpu.sync_copy(i_hbm.at[pl.ds(off, BLK)], idx_v.at[0])
        pltpu.sync_copy(e2_hbm.at[pl.ds(off, BLK)], rows_v.at[0])
        pltpu.sync_copy(rows_v.at[0], acc.at[idx_v.at[0]], add=True)
        plsc.subcore_barrier()
        pltpu.sync_copy(acc.at[pl.ds(sid * stripe, stripe)],
                        out_hbm.at[core, pl.ds(sid * stripe, stripe)])

    return k(e2, i_idx, zrows)


def _stage_a_body(e1_ref, rbf_ref, wkj_ref, bkj_ref,
                  r1_ref, r2_ref, zkj_ref):
    e1 = e1_ref[...]
    xkj = _swish(_mm(e1, wkj_ref[...]) + bkj_ref[...])
    rbf = (rbf_ref[...] @ r1_ref[...]) @ r2_ref[...]
    zkj_ref[...] = xkj * rbf


def _stage_a(e1, rbf0, p):
    E = e1.shape[0]
    return pl.pallas_call(
        _stage_a_body,
        grid=(E // BE,),
        in_specs=[
            pl.BlockSpec((BE, HID), lambda i: (i, 0)),
            pl.BlockSpec((BE, NRAD), lambda i: (i, 0)),
            _full((HID, HID)), _full((1, HID)),
            _full((NRAD, BASIS)), _full((BASIS, HID)),
        ],
        out_specs=pl.BlockSpec((BE, HID), lambda i: (i, 0)),
        out_shape=jax.ShapeDtypeStruct((E, HID), jnp.float32),
        interpret=_INTERPRET,
    )(e1, rbf0,
      p['lin_kj_w'], p['lin_kj_b'][None, :],
      p['lin_rbf1'], p['lin_rbf2'])


def _stage_t_body(xg_ref, dall_ref, ang_ref, wdn_ref, s1_ref, s2_ref,
                  wup_ref, y_ref):
    d = dall_ref[:, :1] / CUTOFF
    rbf = _envelope(d) * _sin_ladder(d)
    ca = jnp.cos(ang_ref[...])
    ts = [jnp.ones_like(ca), ca]
    for _ in range(NSPH - 2):
        ts.append(2.0 * ca * ts[-1] - ts[-2])
    sbf42 = jnp.concatenate([t * rbf for t in ts], axis=1)
    sb8 = _mm(sbf42, s1_ref[...])
    xdn = _swish(_mm(xg_ref[...], wdn_ref[...]))
    y_ref[...] = _mm(xdn * _mm(sb8, s2_ref[...]), wup_ref[...])


def _stage_t(xg, dall, angle, p):
    T = xg.shape[0]
    return pl.pallas_call(
        _stage_t_body,
        grid=(T // BT,),
        in_specs=[
            pl.BlockSpec((BT, HID), lambda i: (i, 0)),
            pl.BlockSpec((BT, HID), lambda i: (i, 0)),
            pl.BlockSpec((BT, 1), lambda i: (i, 0)),
            _full((HID, INT_EMB)),
            _full((NSPH * NRAD, BASIS)),
            _full((BASIS, INT_EMB)),
            _full((INT_EMB, HID)),
        ],
        out_specs=pl.BlockSpec((BT, HID), lambda i: (i, 0)),
        out_shape=jax.ShapeDtypeStruct((T, HID), jnp.float32),
        interpret=_INTERPRET,
    )(xg, dall, angle[:, None], p['lin_down'], p['lin_sbf1'], p['lin_sbf2'],
      p['lin_up'])


def _stage_s_body(xsa_ref, xsb_ref, rbf_ref, e1_ref,
                  wji, bji, wcat, bw1, bb1, bw2, bb2, lw, lb,
                  a1w1, a1b1, a1w2, a1b2, a2w1, a2b1, a2w2, a2b2, lrbf,
                  e1o_ref, e2o_ref):
    e1 = e1_ref[...]
    xji = _swish(_mm(e1, wji[...]) + bji[...])
    xkj = _swish(xsa_ref[0] + xsb_ref[0])
    e = _swish(_mm(xji + xkj, wcat[...]))
    e = e + _swish(_mm(_swish(_mm(e, bw1[...]) + bb1[...]), bw2[...]) + bb2[...])
    e = _swish(_mm(e, lw[...]) + lb[...]) + e1
    e = e + _swish(_mm(_swish(_mm(e, a1w1[...]) + a1b1[...]), a1w2[...]) + a1b2[...])
    e = e + _swish(_mm(_swish(_mm(e, a2w1[...]) + a2b1[...]), a2w2[...]) + a2b2[...])
    e1o_ref[...] = e
    e2o_ref[...] = (rbf_ref[...] @ lrbf[...]) * e


def _stage_s(xs, rbf0, e1, p):
    E = e1.shape[0]
    b = p['before'][0]
    a1, a2 = p['after']
    return pl.pallas_call(
        _stage_s_body,
        grid=(E // BE,),
        in_specs=[
            pl.BlockSpec((1, BE, HID), lambda i: (0, i, 0)),
            pl.BlockSpec((1, BE, HID), lambda i: (1, i, 0)),
            pl.BlockSpec((BE, NRAD), lambda i: (i, 0)),
            pl.BlockSpec((BE, HID), lambda i: (i, 0)),
            _full((HID, HID)), _full((1, HID)), _full((HID, HID)),
            _full((HID, HID)), _full((1, HID)), _full((HID, HID)), _full((1, HID)),
            _full((HID, HID)), _full((1, HID)),
            _full((HID, HID)), _full((1, HID)), _full((HID, HID)), _full((1, HID)),
            _full((HID, HID)), _full((1, HID)), _full((HID, HID)), _full((1, HID)),
            _full((NRAD, HID)),
        ],
        out_specs=[
            pl.BlockSpec((BE, HID), lambda i: (i, 0)),
            pl.BlockSpec((BE, HID), lambda i: (i, 0)),
        ],
        out_shape=[
            jax.ShapeDtypeStruct((E, HID), jnp.float32),
            jax.ShapeDtypeStruct((E, HID), jnp.float32),
        ],
        interpret=_INTERPRET,
    )(xs, xs, rbf0, e1,
      p['lin_ji_w'], p['lin_ji_b'][None, :], p['lin_cat'],
      b['w1'], b['b1'][None, :], b['w2'], b['b2'][None, :],
      p['lin_w'], p['lin_b'][None, :],
      a1['w1'], a1['b1'][None, :], a1['w2'], a1['b2'][None, :],
      a2['w1'], a2['b1'][None, :], a2['w2'], a2['b2'][None, :],
      p['lin_rbf'])


def _upd_v_body(va_ref, vb_ref, batch_ref, upw, upb, l1w, l1b, l2w, l2b,
                l3w, l3b, outw, u_ref):
    v = _mm(va_ref[0] + vb_ref[0], upw[...]) + upb[...]
    v = _swish(_mm(v, l1w[...]) + l1b[...])
    v = _swish(_mm(v, l2w[...]) + l2b[...])
    v = _swish(_mm(v, l3w[...]) + l3b[...])
    vv = v @ outw[...]
    gid = lax.broadcasted_iota(jnp.int32, (N_G, vv.shape[0]), 0)
    onehot = (gid == batch_ref[0]).astype(jnp.float32)

    @pl.when(pl.program_id(0) == 0)
    def _():
        u_ref[...] = jnp.zeros_like(u_ref)

    u_ref[...] += onehot @ vv


def _upd_v(na, batch_pad3, q):
    ls = q['lins']
    return pl.pallas_call(
        _upd_v_body,
        grid=(NPAD // BNV,),
        in_specs=[
            pl.BlockSpec((1, BNV, HID), lambda i: (0, i, 0)),
            pl.BlockSpec((1, BNV, HID), lambda i: (1, i, 0)),
            pl.BlockSpec((1, 1, BNV), lambda i: (i, 0, 0)),
            _full((HID, OUT_EMB)), _full((1, OUT_EMB)),
            _full((OUT_EMB, OUT_EMB)), _full((1, OUT_EMB)),
            _full((OUT_EMB, OUT_EMB)), _full((1, OUT_EMB)),
            _full((OUT_EMB, OUT_EMB)), _full((1, OUT_EMB)),
            _full((OUT_EMB, 1)),
        ],
        out_specs=_full((N_G, 1)),
        out_shape=jax.ShapeDtypeStruct((N_G, 1), jnp.float32),
        interpret=_INTERPRET,
    )(na, na, batch_pad3, q['up_w'], q['up_b'][None, :],
      ls[0]['w'], ls[0]['b'][None, :],
      ls[1]['w'], ls[1]['b'][None, :],
      ls[2]['w'], ls[2]['b'][None, :],
      q['out_w'])


def kernel(x, edge_attr, dist, angle, params, idx_kj, idx_ji, i, batch):
    N = x.shape[0]
    E = edge_attr.shape[0]
    T = angle.shape[0]

    sorted_ji, sorted_kj, angle_st = lax.sort((idx_ji, idx_kj, angle),
                                              num_keys=1)
    starts = jnp.searchsorted(
        sorted_ji, jnp.arange(NCHUNK + 1, dtype=jnp.int32) * WIN).astype(jnp.int32)
    s_al = starts[:-1] & ~7
    e_raw = starts[1:]
    bounds = jnp.stack(
        [s_al, e_raw] + [jnp.zeros((NCHUNK,), jnp.int32)] * 14, axis=1)
    pad = TPAD - T
    idx_kj2d = jnp.concatenate(
        [sorted_kj, jnp.zeros((pad,), jnp.int32)]).reshape(TPAD // 128, 128)
    idx_ji_pad = jnp.concatenate(
        [sorted_ji, jnp.full((pad,), E, jnp.int32)])
    angle_s = jnp.concatenate([angle_st, jnp.zeros((pad,), jnp.float32)])
    dist128 = jnp.pad(dist[:, None] + 0.0, ((0, 0), (0, 127)),
                      constant_values=1.0)
    batch_pad3 = jnp.concatenate(
        [batch, jnp.full((NPAD - N,), N_G, jnp.int32)]).reshape(
            NPAD // BNV, 1, BNV)
    zrows_t = jnp.zeros((WACC // NSUB // 3, HID), jnp.float32)
    zrows_n = jnp.zeros((NPAD // NSUB, HID), jnp.float32)

    e1, rbf0 = _edge_init(edge_attr, dist, params['lin_edge'])
    u = _node_init(x, batch, params['lin_node'])
    dall = _sc_gather(dist128, idx_kj2d)

    for pe, pv in zip(params['update_es'], params['update_vs']):
        zkj = _stage_a(e1, rbf0, pe)
        xg = _sc_gather(zkj, idx_kj2d)
        y = _stage_t(xg, dall, angle_s, pe)
        xs = _sc_tri_scatter(y, idx_ji_pad, bounds, zrows_t)
        e1, e2 = _stage_s(xs, rbf0, e1, pe)
        na = _sc_node_scatter(e2, i, zrows_n)
        u = u + _upd_v(na, batch_pad3, pv)
    return u


# speedup vs baseline: 1.9760x; 1.7095x over previous
import functools

import jax
import jax.numpy as jnp
import numpy as np
from jax import lax
from jax.experimental import pallas as pl
from jax.experimental.pallas import tpu as pltpu
from jax.experimental.pallas import tpu_sc as plsc

NC = 2
NSUB = 16
NW = NC * NSUB

CUTOFF = 5.0
NRAD = 6
NSPH = 7
ENV_P = 5
HID = 128
INT_EMB = 64
BASIS = 8
OUT_EMB = 256
N_G = 64

BE = 2000
BT = 4096
BN = 2000
BNV = 2048

TPAD = 655360
NPAD = 10240
WIN = 8000
NCHUNK = 40
WACC = 8064

_INTERPRET = False


def _swish(x):
    return x / (1.0 + jnp.exp(-x))


def _mm(a, b):
    return jnp.dot(a.astype(jnp.bfloat16), b.astype(jnp.bfloat16),
                   preferred_element_type=jnp.float32)


def _envelope(d):
    p = ENV_P
    a = -(p + 1) * (p + 2) / 2.0
    b = float(p * (p + 2))
    c = -p * (p + 1) / 2.0
    d2 = d * d
    d4 = d2 * d2
    d5 = d4 * d
    return 1.0 / d + a * d5 + b * d5 * d + c * d5 * d2


def _sin_ladder(d):
    s1 = jnp.sin(jnp.pi * d)
    c1 = jnp.cos(jnp.pi * d)
    sins = [s1]
    ck = c1
    for _ in range(NRAD - 1):
        sk = sins[-1]
        sins.append(sk * c1 + ck * s1)
        ck = ck * c1 - sk * s1
    return jnp.concatenate(sins, axis=1)


def _full(spec_shape):
    return pl.BlockSpec(spec_shape, lambda *_: tuple(0 for _ in spec_shape))


def _edge_init_body(ea_ref, dist_ref, we_ref, e1_ref, rbf_ref):
    e1_ref[...] = ea_ref[...] @ we_ref[...]
    d = dist_ref[...] / CUTOFF
    rbf_ref[...] = _envelope(d) * _sin_ladder(d)


def _edge_init(edge_attr, dist, we):
    E = edge_attr.shape[0]
    grid = (E // BE,)
    return pl.pallas_call(
        _edge_init_body,
        grid=grid,
        in_specs=[
            pl.BlockSpec((BE, 12), lambda i: (i, 0)),
            pl.BlockSpec((BE, 1), lambda i: (i, 0)),
            _full((12, HID)),
        ],
        out_specs=[
            pl.BlockSpec((BE, HID), lambda i: (i, 0)),
            pl.BlockSpec((BE, NRAD), lambda i: (i, 0)),
        ],
        out_shape=[
            jax.ShapeDtypeStruct((E, HID), jnp.float32),
            jax.ShapeDtypeStruct((E, NRAD), jnp.float32),
        ],
        interpret=_INTERPRET,
    )(edge_attr, dist[:, None], we)


def _node_init_body(x_ref, batch_ref, wn_ref, u_ref):
    v = x_ref[...] @ wn_ref[...]
    gid = lax.broadcasted_iota(jnp.int32, (N_G, v.shape[0]), 0)
    onehot = (gid == batch_ref[0]).astype(jnp.float32)

    @pl.when(pl.program_id(0) == 0)
    def _():
        u_ref[...] = jnp.zeros_like(u_ref)

    u_ref[...] += onehot @ v


def _node_init(x, batch, wn):
    N = x.shape[0]
    return pl.pallas_call(
        _node_init_body,
        grid=(N // BN,),
        in_specs=[
            pl.BlockSpec((BN, 48), lambda i: (i, 0)),
            pl.BlockSpec((1, 1, BN), lambda i: (i, 0, 0)),
            _full((48, HID)),
        ],
        out_specs=_full((N_G, HID)),
        out_shape=jax.ShapeDtypeStruct((N_G, HID), jnp.float32),
        interpret=_INTERPRET,
    )(x, batch.reshape(N // BN, 1, BN), wn)


_MESH = plsc.VectorSubcoreMesh(core_axis_name="c", subcore_axis_name="s",
                               num_cores=NC, num_subcores=NSUB)


def _sc_gather(xdn, idx_kj2d):
    per_tile = TPAD // NW
    ngrp = per_tile // (4 * 128)

    @functools.partial(
        pl.kernel,
        out_type=jax.ShapeDtypeStruct((TPAD, HID), jnp.float32),
        mesh=_MESH,
        scratch_types=[
            pltpu.VMEM((4, 128), jnp.int32),
            pltpu.VMEM((4, 128, HID), jnp.float32),
            pltpu.SemaphoreType.DMA, pltpu.SemaphoreType.DMA,
        ],
    )
    def k(xdn_hbm, kj_hbm, xg_hbm, idx_v, rows_v, gsem, ssem):
        wid = lax.axis_index("c") * NSUB + lax.axis_index("s")
        grp0 = wid * (ngrp * 4)

        def body(j, _):
            blk = grp0 + j * 4
            pltpu.sync_copy(kj_hbm.at[pl.ds(blk, 4)], idx_v)

            @pl.when(j > 0)
            def _():
                for b in range(4):
                    pltpu.make_async_copy(
                        xg_hbm.at[pl.ds(0, 128)], rows_v.at[b], ssem).wait()

            descs = [pltpu.async_copy(xdn_hbm.at[idx_v.at[b]], rows_v.at[b],
                                      gsem) for b in range(4)]
            for d in descs:
                d.wait()
            for b in range(4):
                pltpu.async_copy(rows_v.at[b],
                                 xg_hbm.at[pl.ds((blk + b) * 128, 128)], ssem)
            return 0

        lax.fori_loop(0, ngrp, body, 0)
        for b in range(4):
            pltpu.make_async_copy(xg_hbm.at[pl.ds(0, 128)], rows_v.at[b],
                                  ssem).wait()

    return k(xdn, idx_kj2d)


def _sc_tri_scatter(y, idx_ji_pad, bounds, zrows):
    zstripe = WACC // NSUB

    @functools.partial(
        pl.kernel,
        out_type=jax.ShapeDtypeStruct((NC, 320000 + 64, HID), jnp.float32),
        mesh=_MESH,
        scratch_types=[
            pltpu.VMEM_SHARED((WACC, HID), jnp.float32),
            pltpu.VMEM((zstripe // 3, HID), jnp.float32),
            pltpu.VMEM((2, 128), jnp.int32),
            pltpu.VMEM((2, 128), jnp.int32),
            pltpu.VMEM((2, 128, HID), jnp.float32),
            pltpu.VMEM((NCHUNK, 16), jnp.int32),
            pltpu.SemaphoreType.DMA,
            pltpu.SemaphoreType.DMA,
        ],
    )
    def k(y_hbm, ji_hbm, bounds_hbm, z_hbm, out_hbm,
          acc, zbuf, idxj_v, idxc_v, rows_v, bounds_v, lsem, ssem):
        core = lax.axis_index("c")
        sid = lax.axis_index("s")
        wid = core * NSUB + sid
        pltpu.sync_copy(z_hbm, zbuf)
        pltpu.sync_copy(bounds_hbm, bounds_v)

        for c in range(NCHUNK):
            for z in range(3):
                pltpu.sync_copy(
                    zbuf,
                    acc.at[pl.ds(sid * zstripe + z * (zstripe // 3),
                                 zstripe // 3)])
            plsc.subcore_barrier()
            row = bounds_v[c]
            s = row[0]
            e = row[1]
            nb = lax.shift_right_logical(e - s + (NW * 128 - 1), 12)
            nb2 = lax.shift_right_logical(nb + 1, 1)
            base = c * WIN

            def inner(g, _, s=s, base=base):
                offs = []
                loads = []
                for p in range(2):
                    off = pl.multiple_of(
                        s + ((2 * g + p) * NW + wid) * 128, 8)
                    offs.append(off)
                    loads.append(pltpu.async_copy(
                        ji_hbm.at[pl.ds(off, 128)], idxj_v.at[p], lsem))
                    loads.append(pltpu.async_copy(
                        y_hbm.at[pl.ds(off, 128)], rows_v.at[p], lsem))
                scats = []
                for p in range(2):
                    loads[2 * p].wait()
                    loads[2 * p + 1].wait()
                    for m in range(8):
                        lv = idxj_v[p, pl.ds(m * 16, 16)] - base
                        inb = (lv >= 0) & (lv < WIN)
                        idxc_v[p, pl.ds(m * 16, 16)] = jnp.where(
                            inb, lv, WIN)
                    scats.append(pltpu.async_copy(
                        rows_v.at[p], acc.at[idxc_v.at[p]], ssem, add=True))
                for sc in scats:
                    sc.wait()
                return 0

            lax.fori_loop(0, nb2, inner, 0)
            plsc.subcore_barrier()
            pltpu.sync_copy(
                acc.at[pl.ds(sid * zstripe, zstripe)],
                out_hbm.at[core, pl.ds(base + sid * zstripe, zstripe)])
            plsc.subcore_barrier()

    return k(y, idx_ji_pad, bounds, zrows)


def _sc_node_scatter(e2, i_idx, zrows):
    E = e2.shape[0]
    per_core = E // NC
    per_tile = per_core // NSUB
    BLK = 80
    nblk = per_tile // BLK
    stripe = NPAD // NSUB

    @functools.partial(
        pl.kernel,
        out_type=jax.ShapeDtypeStruct((NC, NPAD, HID), jnp.float32),
        mesh=_MESH,
        scratch_types=[
            pltpu.VMEM_SHARED((NPAD, HID), jnp.float32),
            pltpu.VMEM((2, BLK), jnp.int32),
            pltpu.VMEM((2, BLK, HID), jnp.float32),
            pltpu.SemaphoreType.DMA,
            pltpu.SemaphoreType.DMA,
        ],
    )
    def k(e2_hbm, i_hbm, z_hbm, out_hbm, acc, idx_v, rows_v, lsem, ssem):
        core = lax.axis_index("c")
        sid = lax.axis_index("s")
        pltpu.sync_copy(z_hbm, acc.at[pl.ds(sid * stripe, stripe)])
        plsc.subcore_barrier()
        base = (core * NSUB + sid) * per_tile

        def pair(g, _):
            loads = []
            for p in range(2):
                off = base + (2 * g + p) * BLK
                loads.append(pltpu.async_copy(
                    i_hbm.at[pl.ds(off, BLK)], idx_v.at[p], lsem))
                loads.append(pltpu.async_copy(
                    e2_hbm.at[pl.ds(off, BLK)], rows_v.at[p], lsem))
            scats = []
            for p in range(2):
                loads[2 * p].wait()
                loads[2 * p + 1].wait()
                scats.append(pltpu.async_copy(
                    rows_v.at[p], acc.at[idx_v.at[p]], ssem, add=True))
            for sc in scats:
                sc.wait()
            return 0

        lax.fori_loop(0, nblk // 2, pair, 0)
        off = base + (nblk - 1) * BLK
        pltpu.sync_copy(i_hbm.at[pl.ds(off, BLK)], idx_v.at[0])
        pltpu.sync_copy(e2_hbm.at[pl.ds(off, BLK)], rows_v.at[0])
        pltpu.sync_copy(rows_v.at[0], acc.at[idx_v.at[0]], add=True)
        plsc.subcore_barrier()
        pltpu.sync_copy(acc.at[pl.ds(sid * stripe, stripe)],
                        out_hbm.at[core, pl.ds(sid * stripe, stripe)])

    return k(e2, i_idx, zrows)


def _stage_a_body(e1_ref, rbf_ref, wkj_ref, bkj_ref,
                  r1_ref, r2_ref, zkj_ref):
    e1 = e1_ref[...]
    xkj = _swish(_mm(e1, wkj_ref[...]) + bkj_ref[...])
    rbf = (rbf_ref[...] @ r1_ref[...]) @ r2_ref[...]
    zkj_ref[...] = xkj * rbf


def _stage_a(e1, rbf0, p):
    E = e1.shape[0]
    return pl.pallas_call(
        _stage_a_body,
        grid=(E // BE,),
        in_specs=[
            pl.BlockSpec((BE, HID), lambda i: (i, 0)),
            pl.BlockSpec((BE, NRAD), lambda i: (i, 0)),
            _full((HID, HID)), _full((1, HID)),
            _full((NRAD, BASIS)), _full((BASIS, HID)),
        ],
        out_specs=pl.BlockSpec((BE, HID), lambda i: (i, 0)),
        out_shape=jax.ShapeDtypeStruct((E, HID), jnp.float32),
        interpret=_INTERPRET,
    )(e1, rbf0,
      p['lin_kj_w'], p['lin_kj_b'][None, :],
      p['lin_rbf1'], p['lin_rbf2'])


def _sbf_body(dall_ref, ang_ref, sbf_ref):
    d = dall_ref[:, :1] / CUTOFF
    rbf = _envelope(d) * _sin_ladder(d)
    ca = jnp.cos(ang_ref[...])
    ts = [jnp.ones_like(ca), ca]
    for _ in range(NSPH - 2):
        ts.append(2.0 * ca * ts[-1] - ts[-2])
    sbf_ref[...] = jnp.concatenate([t * rbf for t in ts], axis=1)


def _sbf_once(dall, angle):
    T = dall.shape[0]
    return pl.pallas_call(
        _sbf_body,
        grid=(T // BT,),
        in_specs=[
            pl.BlockSpec((BT, HID), lambda i: (i, 0)),
            pl.BlockSpec((BT, 1), lambda i: (i, 0)),
        ],
        out_specs=pl.BlockSpec((BT, NSPH * NRAD), lambda i: (i, 0)),
        out_shape=jax.ShapeDtypeStruct((T, NSPH * NRAD), jnp.float32),
        interpret=_INTERPRET,
    )(dall, angle[:, None])


def _stage_t_body(xg_ref, sbf_ref, wdn_ref, s1_ref, s2_ref,
                  wup_ref, y_ref):
    sb8 = _mm(sbf_ref[...], s1_ref[...])
    xdn = _swish(_mm(xg_ref[...], wdn_ref[...]))
    y_ref[...] = _mm(xdn * _mm(sb8, s2_ref[...]), wup_ref[...])


def _stage_t(xg, sbf42, p):
    T = xg.shape[0]
    return pl.pallas_call(
        _stage_t_body,
        grid=(T // BT,),
        in_specs=[
            pl.BlockSpec((BT, HID), lambda i: (i, 0)),
            pl.BlockSpec((BT, NSPH * NRAD), lambda i: (i, 0)),
            _full((HID, INT_EMB)),
            _full((NSPH * NRAD, BASIS)),
            _full((BASIS, INT_EMB)),
            _full((INT_EMB, HID)),
        ],
        out_specs=pl.BlockSpec((BT, HID), lambda i: (i, 0)),
        out_shape=jax.ShapeDtypeStruct((T, HID), jnp.float32),
        interpret=_INTERPRET,
    )(xg, sbf42, p['lin_down'], p['lin_sbf1'], p['lin_sbf2'],
      p['lin_up'])


def _stage_s_body(xsa_ref, xsb_ref, rbf_ref, e1_ref,
                  wji, bji, wcat, bw1, bb1, bw2, bb2, lw, lb,
                  a1w1, a1b1, a1w2, a1b2, a2w1, a2b1, a2w2, a2b2, lrbf,
                  e1o_ref, e2o_ref):
    e1 = e1_ref[...]
    xji = _swish(_mm(e1, wji[...]) + bji[...])
    xkj = _swish(xsa_ref[0] + xsb_ref[0])
    e = _swish(_mm(xji + xkj, wcat[...]))
    e = e + _swish(_mm(_swish(_mm(e, bw1[...]) + bb1[...]), bw2[...]) + bb2[...])
    e = _swish(_mm(e, lw[...]) + lb[...]) + e1
    e = e + _swish(_mm(_swish(_mm(e, a1w1[...]) + a1b1[...]), a1w2[...]) + a1b2[...])
    e = e + _swish(_mm(_swish(_mm(e, a2w1[...]) + a2b1[...]), a2w2[...]) + a2b2[...])
    e1o_ref[...] = e
    e2o_ref[...] = (rbf_ref[...] @ lrbf[...]) * e


def _stage_s(xs, rbf0, e1, p):
    E = e1.shape[0]
    b = p['before'][0]
    a1, a2 = p['after']
    return pl.pallas_call(
        _stage_s_body,
        grid=(E // BE,),
        in_specs=[
            pl.BlockSpec((1, BE, HID), lambda i: (0, i, 0)),
            pl.BlockSpec((1, BE, HID), lambda i: (1, i, 0)),
            pl.BlockSpec((BE, NRAD), lambda i: (i, 0)),
            pl.BlockSpec((BE, HID), lambda i: (i, 0)),
            _full((HID, HID)), _full((1, HID)), _full((HID, HID)),
            _full((HID, HID)), _full((1, HID)), _full((HID, HID)), _full((1, HID)),
            _full((HID, HID)), _full((1, HID)),
            _full((HID, HID)), _full((1, HID)), _full((HID, HID)), _full((1, HID)),
            _full((HID, HID)), _full((1, HID)), _full((HID, HID)), _full((1, HID)),
            _full((NRAD, HID)),
        ],
        out_specs=[
            pl.BlockSpec((BE, HID), lambda i: (i, 0)),
            pl.BlockSpec((BE, HID), lambda i: (i, 0)),
        ],
        out_shape=[
            jax.ShapeDtypeStruct((E, HID), jnp.float32),
            jax.ShapeDtypeStruct((E, HID), jnp.float32),
        ],
        interpret=_INTERPRET,
    )(xs, xs, rbf0, e1,
      p['lin_ji_w'], p['lin_ji_b'][None, :], p['lin_cat'],
      b['w1'], b['b1'][None, :], b['w2'], b['b2'][None, :],
      p['lin_w'], p['lin_b'][None, :],
      a1['w1'], a1['b1'][None, :], a1['w2'], a1['b2'][None, :],
      a2['w1'], a2['b1'][None, :], a2['w2'], a2['b2'][None, :],
      p['lin_rbf'])


def _upd_v_body(va_ref, vb_ref, batch_ref, upw, upb, l1w, l1b, l2w, l2b,
                l3w, l3b, outw, u_ref):
    v = _mm(va_ref[0] + vb_ref[0], upw[...]) + upb[...]
    v = _swish(_mm(v, l1w[...]) + l1b[...])
    v = _swish(_mm(v, l2w[...]) + l2b[...])
    v = _swish(_mm(v, l3w[...]) + l3b[...])
    vv = v @ outw[...]
    gid = lax.broadcasted_iota(jnp.int32, (N_G, vv.shape[0]), 0)
    onehot = (gid == batch_ref[0]).astype(jnp.float32)

    @pl.when(pl.program_id(0) == 0)
    def _():
        u_ref[...] = jnp.zeros_like(u_ref)

    u_ref[...] += onehot @ vv


def _upd_v(na, batch_pad3, q):
    ls = q['lins']
    return pl.pallas_call(
        _upd_v_body,
        grid=(NPAD // BNV,),
        in_specs=[
            pl.BlockSpec((1, BNV, HID), lambda i: (0, i, 0)),
            pl.BlockSpec((1, BNV, HID), lambda i: (1, i, 0)),
            pl.BlockSpec((1, 1, BNV), lambda i: (i, 0, 0)),
            _full((HID, OUT_EMB)), _full((1, OUT_EMB)),
            _full((OUT_EMB, OUT_EMB)), _full((1, OUT_EMB)),
            _full((OUT_EMB, OUT_EMB)), _full((1, OUT_EMB)),
            _full((OUT_EMB, OUT_EMB)), _full((1, OUT_EMB)),
            _full((OUT_EMB, 1)),
        ],
        out_specs=_full((N_G, 1)),
        out_shape=jax.ShapeDtypeStruct((N_G, 1), jnp.float32),
        interpret=_INTERPRET,
    )(na, na, batch_pad3, q['up_w'], q['up_b'][None, :],
      ls[0]['w'], ls[0]['b'][None, :],
      ls[1]['w'], ls[1]['b'][None, :],
      ls[2]['w'], ls[2]['b'][None, :],
      q['out_w'])


def kernel(x, edge_attr, dist, angle, params, idx_kj, idx_ji, i, batch):
    N = x.shape[0]
    E = edge_attr.shape[0]
    T = angle.shape[0]

    sorted_ji, sorted_kj, angle_st = lax.sort((idx_ji, idx_kj, angle),
                                              num_keys=1)
    starts = jnp.searchsorted(
        sorted_ji, jnp.arange(NCHUNK + 1, dtype=jnp.int32) * WIN).astype(jnp.int32)
    s_al = starts[:-1] & ~7
    e_raw = starts[1:]
    bounds = jnp.stack(
        [s_al, e_raw] + [jnp.zeros((NCHUNK,), jnp.int32)] * 14, axis=1)
    pad = TPAD - T
    idx_kj2d = jnp.concatenate(
        [sorted_kj, jnp.zeros((pad,), jnp.int32)]).reshape(TPAD // 128, 128)
    idx_ji_pad = jnp.concatenate(
        [sorted_ji, jnp.full((pad,), E, jnp.int32)])
    angle_s = jnp.concatenate([angle_st, jnp.zeros((pad,), jnp.float32)])
    dist128 = jnp.pad(dist[:, None] + 0.0, ((0, 0), (0, 127)),
                      constant_values=1.0)
    batch_pad3 = jnp.concatenate(
        [batch, jnp.full((NPAD - N,), N_G, jnp.int32)]).reshape(
            NPAD // BNV, 1, BNV)
    zrows_t = jnp.zeros((WACC // NSUB // 3, HID), jnp.float32)
    zrows_n = jnp.zeros((NPAD // NSUB, HID), jnp.float32)

    e1, rbf0 = _edge_init(edge_attr, dist, params['lin_edge'])
    u = _node_init(x, batch, params['lin_node'])
    dall = _sc_gather(dist128, idx_kj2d)
    sbf42 = _sbf_once(dall, angle_s)

    for pe, pv in zip(params['update_es'], params['update_vs']):
        zkj = _stage_a(e1, rbf0, pe)
        xg = _sc_gather(zkj, idx_kj2d)
        y = _stage_t(xg, sbf42, pe)
        xs = _sc_tri_scatter(y, idx_ji_pad, bounds, zrows_t)
        e1, e2 = _stage_s(xs, rbf0, e1, pe)
        na = _sc_node_scatter(e2, i, zrows_n)
        u = u + _upd_v(na, batch_pad3, pv)
    return u
